# baseline (speedup 1.0000x reference)
import functools

import jax
import jax.numpy as jnp
import numpy as np
from jax import lax
from jax.experimental import pallas as pl
from jax.experimental.pallas import tpu as pltpu
from jax.experimental.pallas import tpu_sc as plsc

_NW = 32
_NS = 16
_IG = 80
_CH = 400
_BE = 4000


def _xp_body(x_ref, w_ref, b_ref, o_ref):
    o_ref[...] = (
        jnp.dot(x_ref[...], w_ref[...], preferred_element_type=jnp.float32)
        + b_ref[...]
    )


_SIN_C = (6.2831853038906830, -41.341700855561710, 81.605154770549160,
          -76.703453496313200, 42.029598184164820, -14.913903738035478,
          3.2581807863802723)
_COS_C = (0.9999999999193134, -19.739208758202310, 64.939390113122070,
          -85.456685376067140, 60.242464650163825, -26.406760761349420,
          7.8066075815342780, -1.4609469951322853)


def _poly(q, coeffs):
    acc = jnp.float32(coeffs[-1])
    for c in coeffs[-2::-1]:
        acc = acc * q + jnp.float32(c)
    return acc


def _edge_body(t_ref, xpg_ref, f_ref, ws_ref, wc_ref, a_ref, wh_ref, wv_ref):
    t = t_ref[...]
    u = t * f_ref[...]
    r = u - jnp.round(u)
    q = r * r
    sinv = r * _poly(q, _SIN_C)
    cosv = _poly(q, _COS_C)
    zs = jnp.dot(sinv, ws_ref[...], preferred_element_type=jnp.float32)
    zc = jnp.dot(cosv, wc_ref[...], preferred_element_type=jnp.float32)
    h = jnp.tanh(xpg_ref[...] + zs + zc)
    attn = a_ref[...]
    shift = jnp.sum(jnp.abs(attn))
    s = jnp.sum(h * attn, axis=1, keepdims=True)
    w = jnp.exp(s - shift)
    wh_ref[...] = h * w
    wv_ref[...] = w


def _fin_body(acc_ref, den_ref, o_ref):
    a = acc_ref[0]
    den = jnp.sum(den_ref[...], axis=1, keepdims=True)
    o_ref[...] = jnp.where(den > 0.0, a / den, 0.0)


def _make_gather(n_nodes, n_edges, d):
    epw = n_edges // _NW
    nch = epw // _CH
    nsub = _CH // _IG
    irows = epw // _IG
    mesh = plsc.VectorSubcoreMesh(core_axis_name="c", subcore_axis_name="s")

    @functools.partial(
        pl.kernel,
        out_type=jax.ShapeDtypeStruct((n_edges, d), jnp.float32),
        mesh=mesh,
        scratch_types=[
            pltpu.VMEM((irows, _IG), jnp.int32),
            pltpu.VMEM((2, _CH, d), jnp.float32),
            pltpu.SemaphoreType.DMA,
            pltpu.SemaphoreType.DMA,
        ],
    )
    def gather(xp_hbm, src_hbm, out_hbm, idx_v, rows_v, gsem, wsem):
        wid = lax.axis_index("s") * 2 + lax.axis_index("c")
        base = wid * epw
        pltpu.sync_copy(src_hbm.at[wid], idx_v)

        def fire_g(j, bb):
            for t in range(nsub):
                pltpu.async_copy(
                    xp_hbm.at[idx_v.at[j * nsub + t]],
                    rows_v.at[bb, pl.ds(t * _IG, _IG)],
                    gsem,
                )

        def drain_g(bb):
            for t in range(nsub):
                pltpu.make_async_copy(
                    xp_hbm.at[idx_v.at[t]],
                    rows_v.at[bb, pl.ds(t * _IG, _IG)],
                    gsem,
                ).wait()

        def drain_w(bb):
            pltpu.make_async_copy(
                rows_v.at[bb], out_hbm.at[pl.ds(base, _CH)], wsem
            ).wait()

        fire_g(0, 0)

        @pl.loop(0, nch)
        def _chunk(j):
            bb = j % 2
            drain_g(bb)

            @pl.when(j < nch - 1)
            def _():
                @pl.when(j >= 1)
                def _():
                    drain_w(1 - bb)

                fire_g(j + 1, 1 - bb)

            pltpu.async_copy(
                rows_v.at[bb], out_hbm.at[pl.ds(base + j * _CH, _CH)], wsem
            )

        drain_w(0)
        drain_w(1)

    return gather


def _make_scatter(n_pad, n_edges, d):
    half = n_pad // 2
    accr = half + 256
    rpt = accr // _NS
    zr = 48
    epw = n_edges // _NS
    chs = _IG
    nch = epw // chs
    mesh = plsc.VectorSubcoreMesh(core_axis_name="c", subcore_axis_name="s")

    @functools.partial(
        pl.kernel,
        out_type=(
            jax.ShapeDtypeStruct((2, accr, d), jnp.float32),
            jax.ShapeDtypeStruct((_NW, n_pad), jnp.float32),
        ),
        mesh=mesh,
        scratch_types=[
            pltpu.VMEM((4, _IG), jnp.int32),
            pltpu.VMEM((4, _IG), jnp.int32),
            pltpu.VMEM((4, chs, d), jnp.float32),
            pltpu.VMEM((4, chs), jnp.float32),
            pltpu.VMEM((zr, d), jnp.float32),
            pltpu.VMEM((n_pad,), jnp.float32),
            pltpu.VMEM_SHARED((accr, d), jnp.float32),
            pltpu.SemaphoreType.DMA,
            pltpu.SemaphoreType.DMA,
        ],
        compiler_params=pltpu.CompilerParams(needs_layout_passes=False),
    )
    def scatter(rows_hbm, wv_hbm, dst_hbm, acc_hbm, den_hbm,
                idx_v, idx_s, rows_v, wv_v, zb_v, den_v, acc_sh, lsem, ssem):
        cid = lax.axis_index("c")
        sid = lax.axis_index("s")
        lo = cid * half

        @pl.loop(0, zr)
        def _zrow(r):
            @pl.loop(0, d // 16)
            def _zcol(q):
                zb_v[r, pl.ds(q * 16, 16)] = jnp.zeros((16,), jnp.float32)

        @pl.loop(0, rpt // zr)
        def _zstripe(i):
            pltpu.sync_copy(zb_v, acc_sh.at[pl.ds(sid * rpt + i * zr, zr)])

        @pl.loop(0, n_pad // 16)
        def _zden(i):
            den_v[pl.ds(i * 16, 16)] = jnp.zeros((16,), jnp.float32)

        plsc.subcore_barrier()

        base = sid * epw

        def fire(j, b):
            pltpu.async_copy(dst_hbm.at[sid, j], idx_v.at[b], lsem)
            pltpu.async_copy(
                rows_hbm.at[pl.ds(base + j * chs, chs)], rows_v.at[b], lsem
            )
            pltpu.async_copy(
                wv_hbm.at[pl.ds(base + j * chs, chs)], wv_v.at[b], lsem
            )

        def drain(b):
            pltpu.make_async_copy(dst_hbm.at[sid, 0], idx_v.at[b], lsem).wait()
            pltpu.make_async_copy(
                rows_hbm.at[pl.ds(base, chs)], rows_v.at[b], lsem
            ).wait()
            pltpu.make_async_copy(
                wv_hbm.at[pl.ds(base, chs)], wv_v.at[b], lsem
            ).wait()

        fire(0, 0)

        @pl.loop(0, nch)
        def _chunk(j):
            b = j % 4
            drain(b)

            @pl.when(j < nch - 1)
            def _():
                fire(j + 1, (j + 1) % 4)

            for g in range(chs // 16):
                idx16 = idx_v[b, pl.ds(g * 16, 16)]
                loc = idx16 - lo
                valid = (loc >= 0) & (loc < half)
                idx_s[b, pl.ds(g * 16, 16)] = jnp.where(valid, loc, half)
                w16 = wv_v[b, pl.ds(g * 16, 16)]
                plsc.addupdate_scatter(
                    den_v, [idx16], jnp.where(valid, w16, 0.0)
                )
            pltpu.sync_copy(rows_v.at[b], acc_sh.at[idx_s.at[b]], add=True)

        plsc.subcore_barrier()

        @pl.loop(0, rpt // zr)
        def _out(i):
            pltpu.sync_copy(acc_sh.at[pl.ds(sid * rpt + i * zr, zr)], zb_v)
            pltpu.sync_copy(
                zb_v, acc_hbm.at[cid, pl.ds(sid * rpt + i * zr, zr)]
            )

        wid = sid * 2 + cid

        @pl.loop(0, n_pad // 2048)
        def _dout(k):
            pltpu.sync_copy(
                den_v.at[pl.ds(k * 2048, 2048)],
                den_hbm.at[wid, pl.ds(k * 2048, 2048)],
            )

    return scatter


def kernel(x, edge_index, edge_t, W, b, attn, freqs):
    n, d_in = x.shape
    e = edge_t.shape[0]
    d_out = W.shape[0]
    half_t = freqs.shape[0]

    src = edge_index[0].astype(jnp.int32)
    dst = edge_index[1].astype(jnp.int32)
    wxt = W[:, :d_in].T
    wst = W[:, d_in:d_in + half_t].T
    wct = W[:, d_in + half_t:].T
    b2 = b.reshape(1, d_out)
    f2 = freqs.reshape(1, half_t)
    t2 = edge_t.reshape(e, 1)
    src2 = src.reshape(_NW, e // (_NW * _IG), _IG)
    dst2 = dst.reshape(_NS, e // (_NS * _IG), _IG)

    nb = 5
    xp = pl.pallas_call(
        _xp_body,
        grid=(nb,),
        in_specs=[
            pl.BlockSpec((n // nb, d_in), lambda i: (i, 0)),
            pl.BlockSpec((d_in, d_out), lambda i: (0, 0)),
            pl.BlockSpec((1, d_out), lambda i: (0, 0)),
        ],
        out_specs=pl.BlockSpec((n // nb, d_out), lambda i: (i, 0)),
        out_shape=jax.ShapeDtypeStruct((n, d_out), jnp.float32),
    )(x, wxt, b2)

    xpg = _make_gather(n, e, d_out)(xp, src2)

    geb = e // _BE
    rows, wv = pl.pallas_call(
        _edge_body,
        grid=(geb,),
        in_specs=[
            pl.BlockSpec((_BE, 1), lambda i: (i, 0)),
            pl.BlockSpec((_BE, d_out), lambda i: (i, 0)),
            pl.BlockSpec((1, half_t), lambda i: (0, 0)),
            pl.BlockSpec((half_t, d_out), lambda i: (0, 0)),
            pl.BlockSpec((half_t, d_out), lambda i: (0, 0)),
            pl.BlockSpec((1, d_out), lambda i: (0, 0)),
        ],
        out_specs=[
            pl.BlockSpec((_BE, d_out), lambda i: (i, 0)),
            pl.BlockSpec((_BE, 1), lambda i: (i, 0)),
        ],
        out_shape=[
            jax.ShapeDtypeStruct((e, d_out), jnp.float32),
            jax.ShapeDtypeStruct((e, 1), jnp.float32),
        ],
    )(t2, xpg, f2, wst, wct, attn)

    n_pad = ((n + 2047) // 2048) * 2048
    acc, dens = _make_scatter(n_pad, e, d_out)(rows, wv.reshape(e), dst2)

    fr = 640
    fb = n_pad // fr
    bpc = (n_pad // 2) // fr
    out = pl.pallas_call(
        _fin_body,
        grid=(fb,),
        in_specs=[
            pl.BlockSpec((1, fr, d_out), lambda i: (i // bpc, i % bpc, 0)),
            pl.BlockSpec((fr, _NW), lambda i: (i, 0)),
        ],
        out_specs=pl.BlockSpec((fr, d_out), lambda i: (i, 0)),
        out_shape=jax.ShapeDtypeStruct((n, d_out), jnp.float32),
    )(acc, dens.T)
    return out


# speedup vs baseline: 1.0742x; 1.0742x over previous
import functools

import jax
import jax.numpy as jnp
import numpy as np
from jax import lax
from jax.experimental import pallas as pl
from jax.experimental.pallas import tpu as pltpu
from jax.experimental.pallas import tpu_sc as plsc

_NW = 32
_NS = 16
_IG = 80
_CH = 400
_BE = 4000


def _xp_body(x_ref, w_ref, b_ref, o_ref):
    o_ref[...] = (
        jnp.dot(x_ref[...], w_ref[...], preferred_element_type=jnp.float32)
        + b_ref[...]
    )


_SIN_C = (6.2831853038906830, -41.341700855561710, 81.605154770549160,
          -76.703453496313200, 42.029598184164820, -14.913903738035478,
          3.2581807863802723)
_COS_C = (0.9999999999193134, -19.739208758202310, 64.939390113122070,
          -85.456685376067140, 60.242464650163825, -26.406760761349420,
          7.8066075815342780, -1.4609469951322853)


def _poly(q, coeffs):
    acc = jnp.float32(coeffs[-1])
    for c in coeffs[-2::-1]:
        acc = acc * q + jnp.float32(c)
    return acc


def _edge_body(t_ref, xpg_ref, f_ref, ws_ref, wc_ref, a_ref, wh_ref, wv_ref):
    t = t_ref[...]
    u = t * f_ref[...]
    r = u - jnp.round(u)
    q = r * r
    sinv = r * _poly(q, _SIN_C)
    cosv = _poly(q, _COS_C)
    zs = jnp.dot(sinv, ws_ref[...], preferred_element_type=jnp.float32)
    zc = jnp.dot(cosv, wc_ref[...], preferred_element_type=jnp.float32)
    h = jnp.tanh(xpg_ref[...] + zs + zc)
    attn = a_ref[...]
    shift = jnp.sum(jnp.abs(attn))
    s = jnp.sum(h * attn, axis=1, keepdims=True)
    w = jnp.exp(s - shift)
    wh_ref[...] = h * w
    wv_ref[...] = w


def _fin_body(acc0_ref, acc1_ref, den0_ref, den1_ref, o_ref):
    a = acc0_ref[0] + acc1_ref[0]
    den = (jnp.sum(den0_ref[...], axis=1, keepdims=True)
           + jnp.sum(den1_ref[...], axis=1, keepdims=True))
    o_ref[...] = jnp.where(den > 0.0, a / den, 0.0)


def _make_gather(n_nodes, n_edges, d, dtype):
    epw = n_edges // _NW
    nch = epw // _CH
    nsub = _CH // _IG
    irows = epw // _IG
    mesh = plsc.VectorSubcoreMesh(core_axis_name="c", subcore_axis_name="s")

    @functools.partial(
        pl.kernel,
        out_type=jax.ShapeDtypeStruct((n_edges, d), dtype),
        mesh=mesh,
        scratch_types=[
            pltpu.VMEM((irows, _IG), jnp.int32),
            pltpu.VMEM((2, _CH, d), dtype),
            pltpu.SemaphoreType.DMA,
            pltpu.SemaphoreType.DMA,
        ],
    )
    def gather(xp_hbm, src_hbm, out_hbm, idx_v, rows_v, gsem, wsem):
        wid = lax.axis_index("s") * 2 + lax.axis_index("c")
        base = wid * epw
        pltpu.sync_copy(src_hbm.at[wid], idx_v)

        def fire_g(j, bb):
            for t in range(nsub):
                pltpu.async_copy(
                    xp_hbm.at[idx_v.at[j * nsub + t]],
                    rows_v.at[bb, pl.ds(t * _IG, _IG)],
                    gsem,
                )

        def drain_g(bb):
            for t in range(nsub):
                pltpu.make_async_copy(
                    xp_hbm.at[idx_v.at[t]],
                    rows_v.at[bb, pl.ds(t * _IG, _IG)],
                    gsem,
                ).wait()

        def drain_w(bb):
            pltpu.make_async_copy(
                rows_v.at[bb], out_hbm.at[pl.ds(base, _CH)], wsem
            ).wait()

        fire_g(0, 0)

        @pl.loop(0, nch)
        def _chunk(j):
            bb = j % 2
            drain_g(bb)

            @pl.when(j < nch - 1)
            def _():
                @pl.when(j >= 1)
                def _():
                    drain_w(1 - bb)

                fire_g(j + 1, 1 - bb)

            pltpu.async_copy(
                rows_v.at[bb], out_hbm.at[pl.ds(base + j * _CH, _CH)], wsem
            )

        drain_w(0)
        drain_w(1)

    return gather


def _make_scatter(n_pad, n_edges, d):
    half = n_pad // 2
    accr = half + 256
    rpt = accr // _NS
    zr = 48
    epw = n_edges // _NS
    chs = _IG
    nch = epw // chs
    mesh = plsc.VectorSubcoreMesh(core_axis_name="c", subcore_axis_name="s")

    @functools.partial(
        pl.kernel,
        out_type=(
            jax.ShapeDtypeStruct((2, accr, d), jnp.float32),
            jax.ShapeDtypeStruct((_NW, n_pad), jnp.float32),
        ),
        mesh=mesh,
        scratch_types=[
            pltpu.VMEM((4, _IG), jnp.int32),
            pltpu.VMEM((4, _IG), jnp.int32),
            pltpu.VMEM((4, chs, d), jnp.float32),
            pltpu.VMEM((4, chs), jnp.float32),
            pltpu.VMEM((zr, d), jnp.float32),
            pltpu.VMEM((n_pad,), jnp.float32),
            pltpu.VMEM_SHARED((accr, d), jnp.float32),
            pltpu.SemaphoreType.DMA,
            pltpu.SemaphoreType.DMA,
        ],
        compiler_params=pltpu.CompilerParams(needs_layout_passes=False),
    )
    def scatter(rows_hbm, wv_hbm, dst_hbm, acc_hbm, den_hbm,
                idx_v, idx_s, rows_v, wv_v, zb_v, den_v, acc_sh, lsem, ssem):
        cid = lax.axis_index("c")
        sid = lax.axis_index("s")
        lo = cid * half

        @pl.loop(0, zr)
        def _zrow(r):
            @pl.loop(0, d // 16)
            def _zcol(q):
                zb_v[r, pl.ds(q * 16, 16)] = jnp.zeros((16,), jnp.float32)

        @pl.loop(0, rpt // zr)
        def _zstripe(i):
            pltpu.sync_copy(zb_v, acc_sh.at[pl.ds(sid * rpt + i * zr, zr)])

        @pl.loop(0, n_pad // 16)
        def _zden(i):
            den_v[pl.ds(i * 16, 16)] = jnp.zeros((16,), jnp.float32)

        plsc.subcore_barrier()

        base = sid * epw

        def fire(j, b):
            pltpu.async_copy(dst_hbm.at[sid, j], idx_v.at[b], lsem)
            pltpu.async_copy(
                rows_hbm.at[pl.ds(base + j * chs, chs)], rows_v.at[b], lsem
            )
            pltpu.async_copy(
                wv_hbm.at[pl.ds(base + j * chs, chs)], wv_v.at[b], lsem
            )

        def drain(b):
            pltpu.make_async_copy(dst_hbm.at[sid, 0], idx_v.at[b], lsem).wait()
            pltpu.make_async_copy(
                rows_hbm.at[pl.ds(base, chs)], rows_v.at[b], lsem
            ).wait()
            pltpu.make_async_copy(
                wv_hbm.at[pl.ds(base, chs)], wv_v.at[b], lsem
            ).wait()

        fire(0, 0)

        @pl.loop(0, nch)
        def _chunk(j):
            b = j % 4
            drain(b)

            @pl.when(j < nch - 1)
            def _():
                fire(j + 1, (j + 1) % 4)

            for g in range(chs // 16):
                idx16 = idx_v[b, pl.ds(g * 16, 16)]
                loc = idx16 - lo
                valid = (loc >= 0) & (loc < half)
                idx_s[b, pl.ds(g * 16, 16)] = jnp.where(valid, loc, half)
                w16 = wv_v[b, pl.ds(g * 16, 16)]
                plsc.addupdate_scatter(
                    den_v, [idx16], jnp.where(valid, w16, 0.0)
                )
            pltpu.sync_copy(rows_v.at[b], acc_sh.at[idx_s.at[b]], add=True)

        plsc.subcore_barrier()

        @pl.loop(0, rpt // zr)
        def _out(i):
            pltpu.sync_copy(acc_sh.at[pl.ds(sid * rpt + i * zr, zr)], zb_v)
            pltpu.sync_copy(
                zb_v, acc_hbm.at[cid, pl.ds(sid * rpt + i * zr, zr)]
            )

        wid = sid * 2 + cid

        @pl.loop(0, n_pad // 2048)
        def _dout(k):
            pltpu.sync_copy(
                den_v.at[pl.ds(k * 2048, 2048)],
                den_hbm.at[wid, pl.ds(k * 2048, 2048)],
            )

    return scatter


def kernel(x, edge_index, edge_t, W, b, attn, freqs):
    n, d_in = x.shape
    e = edge_t.shape[0]
    d_out = W.shape[0]
    half_t = freqs.shape[0]

    src = edge_index[0].astype(jnp.int32)
    dst = edge_index[1].astype(jnp.int32)
    wxt = W[:, :d_in].T
    wst = W[:, d_in:d_in + half_t].T
    wct = W[:, d_in + half_t:].T
    b2 = b.reshape(1, d_out)
    f2 = freqs.reshape(1, half_t)
    t2 = edge_t.reshape(e, 1)
    src2 = src.reshape(_NW, e // (_NW * _IG), _IG)
    dst2 = dst.reshape(_NS, e // (_NS * _IG), _IG)

    nb = 5
    xp = pl.pallas_call(
        _xp_body,
        grid=(nb,),
        in_specs=[
            pl.BlockSpec((n // nb, d_in), lambda i: (i, 0)),
            pl.BlockSpec((d_in, d_out), lambda i: (0, 0)),
            pl.BlockSpec((1, d_out), lambda i: (0, 0)),
        ],
        out_specs=pl.BlockSpec((n // nb, d_out), lambda i: (i, 0)),
        out_shape=jax.ShapeDtypeStruct((n, d_out), jnp.float32),
    )(x, wxt, b2)

    xpg = _make_gather(n, e, d_out, jnp.float32)(xp, src2)

    n_pad = ((n + 2047) // 2048) * 2048
    e2 = e // 2
    geb = e2 // _BE
    dst3 = dst.reshape(2, _NS, e2 // (_NS * _IG), _IG)
    scatter_call = _make_scatter(n_pad, e2, d_out)
    accs, denss = [], []
    for hh in range(2):
        rows_h, wv_h = pl.pallas_call(
            _edge_body,
            grid=(geb,),
            in_specs=[
                pl.BlockSpec((_BE, 1), lambda i, hh=hh: (i + hh * geb, 0)),
                pl.BlockSpec((_BE, d_out), lambda i, hh=hh: (i + hh * geb, 0)),
                pl.BlockSpec((1, half_t), lambda i: (0, 0)),
                pl.BlockSpec((half_t, d_out), lambda i: (0, 0)),
                pl.BlockSpec((half_t, d_out), lambda i: (0, 0)),
                pl.BlockSpec((1, d_out), lambda i: (0, 0)),
            ],
            out_specs=[
                pl.BlockSpec((_BE, d_out), lambda i: (i, 0)),
                pl.BlockSpec((_BE, 1), lambda i: (i, 0)),
            ],
            out_shape=[
                jax.ShapeDtypeStruct((e2, d_out), jnp.float32),
                jax.ShapeDtypeStruct((e2, 1), jnp.float32),
            ],
        )(t2, xpg, f2, wst, wct, attn)
        acc_h, dens_h = scatter_call(rows_h, wv_h.reshape(e2), dst3[hh])
        accs.append(acc_h)
        denss.append(dens_h)

    fr = 640
    fb = n_pad // fr
    bpc = (n_pad // 2) // fr
    out = pl.pallas_call(
        _fin_body,
        grid=(fb,),
        in_specs=[
            pl.BlockSpec((1, fr, d_out), lambda i: (i // bpc, i % bpc, 0)),
            pl.BlockSpec((1, fr, d_out), lambda i: (i // bpc, i % bpc, 0)),
            pl.BlockSpec((fr, _NW), lambda i: (i, 0)),
            pl.BlockSpec((fr, _NW), lambda i: (i, 0)),
        ],
        out_specs=pl.BlockSpec((fr, d_out), lambda i: (i, 0)),
        out_shape=jax.ShapeDtypeStruct((n, d_out), jnp.float32),
    )(accs[0], accs[1], denss[0].T, denss[1].T)
    return out


# speedup vs baseline: 1.0839x; 1.0090x over previous
import functools

import jax
import jax.numpy as jnp
import numpy as np
from jax import lax
from jax.experimental import pallas as pl
from jax.experimental.pallas import tpu as pltpu
from jax.experimental.pallas import tpu_sc as plsc

_NW = 32
_NS = 16
_IG = 80
_CH = 400
_BE = 4000


def _xp_body(x_ref, w_ref, b_ref, o_ref):
    o_ref[...] = (
        jnp.dot(x_ref[...], w_ref[...], preferred_element_type=jnp.float32)
        + b_ref[...]
    )


_SIN_C = (6.2831853038906830, -41.341700855561710, 81.605154770549160,
          -76.703453496313200, 42.029598184164820, -14.913903738035478,
          3.2581807863802723)
_COS_C = (0.9999999999193134, -19.739208758202310, 64.939390113122070,
          -85.456685376067140, 60.242464650163825, -26.406760761349420,
          7.8066075815342780, -1.4609469951322853)


def _poly(q, coeffs):
    acc = jnp.float32(coeffs[-1])
    for c in coeffs[-2::-1]:
        acc = acc * q + jnp.float32(c)
    return acc


def _edge_body(t_ref, xpg_ref, f_ref, ws_ref, wc_ref, a_ref, wh_ref, wv_ref):
    t = t_ref[...]
    u = t * f_ref[...]
    r = u - jnp.round(u)
    q = r * r
    sinv = r * _poly(q, _SIN_C)
    cosv = _poly(q, _COS_C)
    zs = jnp.dot(sinv, ws_ref[...], preferred_element_type=jnp.float32)
    zc = jnp.dot(cosv, wc_ref[...], preferred_element_type=jnp.float32)
    h = jnp.tanh(xpg_ref[...] + zs + zc)
    attn = a_ref[...]
    shift = jnp.sum(jnp.abs(attn))
    s = jnp.sum(h * attn, axis=1, keepdims=True)
    w = jnp.exp(s - shift)
    wh_ref[...] = h * w
    wv_ref[...] = w


def _fin_body(acc0_ref, acc1_ref, den0_ref, den1_ref, o_ref):
    a = acc0_ref[0] + acc1_ref[0]
    den = (jnp.sum(den0_ref[...], axis=1, keepdims=True)
           + jnp.sum(den1_ref[...], axis=1, keepdims=True))
    o_ref[...] = jnp.where(den > 0.0, a / den, 0.0)


def _make_gather(n_nodes, n_edges, d, dtype, ig, ch):
    epw = n_edges // _NW
    nch = epw // ch
    nsub = ch // ig
    irows = epw // ig
    mesh = plsc.VectorSubcoreMesh(core_axis_name="c", subcore_axis_name="s")

    @functools.partial(
        pl.kernel,
        out_type=jax.ShapeDtypeStruct((n_edges, d), dtype),
        mesh=mesh,
        scratch_types=[
            pltpu.VMEM((irows, ig), jnp.int32),
            pltpu.VMEM((2, ch, d), dtype),
            pltpu.SemaphoreType.DMA,
            pltpu.SemaphoreType.DMA,
        ],
    )
    def gather(xp_hbm, src_hbm, out_hbm, idx_v, rows_v, gsem, wsem):
        wid = lax.axis_index("s") * 2 + lax.axis_index("c")
        base = wid * epw
        pltpu.sync_copy(src_hbm.at[wid], idx_v)

        def fire_g(j, bb):
            for t in range(nsub):
                pltpu.async_copy(
                    xp_hbm.at[idx_v.at[j * nsub + t]],
                    rows_v.at[bb, pl.ds(t * ig, ig)],
                    gsem,
                )

        def drain_g(bb):
            for t in range(nsub):
                pltpu.make_async_copy(
                    xp_hbm.at[idx_v.at[t]],
                    rows_v.at[bb, pl.ds(t * ig, ig)],
                    gsem,
                ).wait()

        def drain_w(bb):
            pltpu.make_async_copy(
                rows_v.at[bb], out_hbm.at[pl.ds(base, ch)], wsem
            ).wait()

        fire_g(0, 0)

        @pl.loop(0, nch)
        def _chunk(j):
            bb = j % 2
            drain_g(bb)

            @pl.when(j < nch - 1)
            def _():
                @pl.when(j >= 1)
                def _():
                    drain_w(1 - bb)

                fire_g(j + 1, 1 - bb)

            pltpu.async_copy(
                rows_v.at[bb], out_hbm.at[pl.ds(base + j * ch, ch)], wsem
            )

        drain_w(0)
        drain_w(1)

    return gather


def _make_scatter(n_pad, n_edges, d):
    half = n_pad // 2
    accr = half + 256
    rpt = accr // _NS
    zr = 48
    epw = n_edges // _NS
    chs = _IG
    nch = epw // chs
    mesh = plsc.VectorSubcoreMesh(core_axis_name="c", subcore_axis_name="s")

    @functools.partial(
        pl.kernel,
        out_type=(
            jax.ShapeDtypeStruct((2, accr, d), jnp.float32),
            jax.ShapeDtypeStruct((_NW, n_pad), jnp.float32),
        ),
        mesh=mesh,
        scratch_types=[
            pltpu.VMEM((4, _IG), jnp.int32),
            pltpu.VMEM((4, _IG), jnp.int32),
            pltpu.VMEM((4, chs, d), jnp.float32),
            pltpu.VMEM((4, chs), jnp.float32),
            pltpu.VMEM((zr, d), jnp.float32),
            pltpu.VMEM((n_pad,), jnp.float32),
            pltpu.VMEM_SHARED((accr, d), jnp.float32),
            pltpu.SemaphoreType.DMA,
            pltpu.SemaphoreType.DMA,
        ],
        compiler_params=pltpu.CompilerParams(needs_layout_passes=False),
    )
    def scatter(rows_hbm, wv_hbm, dst_hbm, acc_hbm, den_hbm,
                idx_v, idx_s, rows_v, wv_v, zb_v, den_v, acc_sh, lsem, ssem):
        cid = lax.axis_index("c")
        sid = lax.axis_index("s")
        lo = cid * half

        @pl.loop(0, zr)
        def _zrow(r):
            @pl.loop(0, d // 16)
            def _zcol(q):
                zb_v[r, pl.ds(q * 16, 16)] = jnp.zeros((16,), jnp.float32)

        @pl.loop(0, rpt // zr)
        def _zstripe(i):
            pltpu.sync_copy(zb_v, acc_sh.at[pl.ds(sid * rpt + i * zr, zr)])

        @pl.loop(0, n_pad // 16)
        def _zden(i):
            den_v[pl.ds(i * 16, 16)] = jnp.zeros((16,), jnp.float32)

        plsc.subcore_barrier()

        base = sid * epw

        def fire(j, b):
            pltpu.async_copy(dst_hbm.at[sid, j], idx_v.at[b], lsem)
            pltpu.async_copy(
                rows_hbm.at[pl.ds(base + j * chs, chs)], rows_v.at[b], lsem
            )
            pltpu.async_copy(
                wv_hbm.at[pl.ds(base + j * chs, chs)], wv_v.at[b], lsem
            )

        def drain(b):
            pltpu.make_async_copy(dst_hbm.at[sid, 0], idx_v.at[b], lsem).wait()
            pltpu.make_async_copy(
                rows_hbm.at[pl.ds(base, chs)], rows_v.at[b], lsem
            ).wait()
            pltpu.make_async_copy(
                wv_hbm.at[pl.ds(base, chs)], wv_v.at[b], lsem
            ).wait()

        fire(0, 0)

        @pl.loop(0, nch)
        def _chunk(j):
            b = j % 4
            drain(b)

            @pl.when(j < nch - 1)
            def _():
                fire(j + 1, (j + 1) % 4)

            for g in range(chs // 16):
                idx16 = idx_v[b, pl.ds(g * 16, 16)]
                loc = idx16 - lo
                valid = (loc >= 0) & (loc < half)
                idx_s[b, pl.ds(g * 16, 16)] = jnp.where(valid, loc, half)
                w16 = wv_v[b, pl.ds(g * 16, 16)]
                plsc.addupdate_scatter(
                    den_v, [idx16], jnp.where(valid, w16, 0.0)
                )
            pltpu.sync_copy(rows_v.at[b], acc_sh.at[idx_s.at[b]], add=True)

        plsc.subcore_barrier()

        @pl.loop(0, rpt // zr)
        def _out(i):
            pltpu.sync_copy(acc_sh.at[pl.ds(sid * rpt + i * zr, zr)], zb_v)
            pltpu.sync_copy(
                zb_v, acc_hbm.at[cid, pl.ds(sid * rpt + i * zr, zr)]
            )

        wid = sid * 2 + cid

        @pl.loop(0, n_pad // 2048)
        def _dout(k):
            pltpu.sync_copy(
                den_v.at[pl.ds(k * 2048, 2048)],
                den_hbm.at[wid, pl.ds(k * 2048, 2048)],
            )

    return scatter


def kernel(x, edge_index, edge_t, W, b, attn, freqs):
    n, d_in = x.shape
    e = edge_t.shape[0]
    d_out = W.shape[0]
    half_t = freqs.shape[0]

    src = edge_index[0].astype(jnp.int32)
    dst = edge_index[1].astype(jnp.int32)
    wxt = W[:, :d_in].T
    wst = W[:, d_in:d_in + half_t].T
    wct = W[:, d_in + half_t:].T
    b2 = b.reshape(1, d_out)
    f2 = freqs.reshape(1, half_t)
    t2 = edge_t.reshape(e, 1)
    gig = 40
    src3 = src.reshape(2, _NW, (e // 2) // (_NW * gig), gig)

    nb = 5
    xp = pl.pallas_call(
        _xp_body,
        grid=(nb,),
        in_specs=[
            pl.BlockSpec((n // nb, d_in), lambda i: (i, 0)),
            pl.BlockSpec((d_in, d_out), lambda i: (0, 0)),
            pl.BlockSpec((1, d_out), lambda i: (0, 0)),
        ],
        out_specs=pl.BlockSpec((n // nb, d_out), lambda i: (i, 0)),
        out_shape=jax.ShapeDtypeStruct((n, d_out), jnp.float32),
    )(x, wxt, b2)

    n_pad = ((n + 2047) // 2048) * 2048
    e2 = e // 2
    geb = e2 // _BE
    dst3 = dst.reshape(2, _NS, e2 // (_NS * _IG), _IG)
    gather_call = _make_gather(n, e2, d_out, jnp.float32, gig, 200)
    scatter_call = _make_scatter(n_pad, e2, d_out)
    xpgs = [gather_call(xp, src3[hh]) for hh in range(2)]
    accs, denss = [], []
    for hh in range(2):
        rows_h, wv_h = pl.pallas_call(
            _edge_body,
            grid=(geb,),
            in_specs=[
                pl.BlockSpec((_BE, 1), lambda i, hh=hh: (i + hh * geb, 0)),
                pl.BlockSpec((_BE, d_out), lambda i: (i, 0)),
                pl.BlockSpec((1, half_t), lambda i: (0, 0)),
                pl.BlockSpec((half_t, d_out), lambda i: (0, 0)),
                pl.BlockSpec((half_t, d_out), lambda i: (0, 0)),
                pl.BlockSpec((1, d_out), lambda i: (0, 0)),
            ],
            out_specs=[
                pl.BlockSpec((_BE, d_out), lambda i: (i, 0)),
                pl.BlockSpec((_BE, 1), lambda i: (i, 0)),
            ],
            out_shape=[
                jax.ShapeDtypeStruct((e2, d_out), jnp.float32),
                jax.ShapeDtypeStruct((e2, 1), jnp.float32),
            ],
        )(t2, xpgs[hh], f2, wst, wct, attn)
        acc_h, dens_h = scatter_call(rows_h, wv_h.reshape(e2), dst3[hh])
        accs.append(acc_h)
        denss.append(dens_h)

    fr = 640
    fb = n_pad // fr
    bpc = (n_pad // 2) // fr
    out = pl.pallas_call(
        _fin_body,
        grid=(fb,),
        in_specs=[
            pl.BlockSpec((1, fr, d_out), lambda i: (i // bpc, i % bpc, 0)),
            pl.BlockSpec((1, fr, d_out), lambda i: (i // bpc, i % bpc, 0)),
            pl.BlockSpec((fr, _NW), lambda i: (i, 0)),
            pl.BlockSpec((fr, _NW), lambda i: (i, 0)),
        ],
        out_specs=pl.BlockSpec((fr, d_out), lambda i: (i, 0)),
        out_shape=jax.ShapeDtypeStruct((n, d_out), jnp.float32),
    )(accs[0], accs[1], denss[0].T, denss[1].T)
    return out
